# baseline (speedup 1.0000x reference)
import functools

import jax
import jax.numpy as jnp
from jax import lax
from jax.experimental import pallas as pl
from jax.experimental.pallas import tpu as pltpu
from jax.experimental.pallas import tpu_sc as plsc

N_NODES = 10000
D_FEAT = 128
AUGD = 144
ROWS = 10112
N_EDGES = 320000
NC, NS = 2, 16
NW = NC * NS
K = 128
CPT = 160
NF = 80
NS1 = CPT - NF
CH = NS * CPT
NEP = CH * K
STRIPE = ROWS // NS

IN_DIM = 2 * D_FEAT
H2 = 192
EMB = 128


@functools.cache
def _make_sc_agg():
    mesh = plsc.VectorSubcoreMesh(
        core_axis_name="c", subcore_axis_name="s",
        num_cores=NC, num_subcores=NS)

    @functools.partial(
        pl.kernel,
        out_type=jax.ShapeDtypeStruct((NC, ROWS, AUGD), jnp.float32),
        mesh=mesh,
        scratch_types=[
            pltpu.VMEM((4, K), jnp.int32),
            pltpu.VMEM((2, K, AUGD), jnp.float32),
            pltpu.VMEM_SHARED((ROWS, AUGD), jnp.float32),
            pltpu.SemaphoreType.DMA((2,)),
            pltpu.SemaphoreType.DMA((2,)),
        ],
        compiler_params=pltpu.CompilerParams(use_tc_tiling_on_sc=False),
    )
    def sc_agg(xaug_hbm, epk_hbm, parts_out, idxb, datab, acc, sg, si):
        c = lax.axis_index("c")
        s = lax.axis_index("s")
        start = jnp.where(c == 0, s * NF, NS * NF + s * NS1)
        n = jnp.where(c == 0, NF, NS1)

        zb = datab.at[0]

        def _zrow(i, _):
            for g in range(AUGD // 16):
                zb[i, pl.ds(g * 16, 16)] = jnp.zeros((16,), jnp.float32)
            return _
        lax.fori_loop(0, K, _zrow, None)
        for kk in range(STRIPE // K):
            pltpu.sync_copy(zb, acc.at[pl.ds(s * STRIPE + kk * K, K)])
        rem = STRIPE % K
        if rem:
            pltpu.sync_copy(zb.at[pl.ds(0, rem)],
                            acc.at[pl.ds(s * STRIPE + (STRIPE // K) * K, rem)])
        plsc.subcore_barrier()

        pltpu.sync_copy(epk_hbm.at[start], idxb.at[pl.ds(0, 2)])
        pltpu.async_copy(xaug_hbm.at[idxb.at[0]], datab.at[0], sg.at[0])

        @pl.when(n > 1)
        def _():
            pltpu.async_copy(epk_hbm.at[start + 1], idxb.at[pl.ds(2, 2)],
                             si.at[1])

        def body(i, _):
            slot = lax.rem(i, 2)
            nxt = 1 - slot
            g = start + i

            @pl.when(i + 1 < n)
            def _():
                pltpu.make_async_copy(epk_hbm.at[g + 1],
                                      idxb.at[pl.ds(nxt * 2, 2)],
                                      si.at[nxt]).wait()
                pltpu.async_copy(xaug_hbm.at[idxb.at[nxt * 2]],
                                 datab.at[nxt], sg.at[nxt])

            pltpu.make_async_copy(xaug_hbm.at[pl.ds(0, K)], datab.at[slot],
                                  sg.at[slot]).wait()
            pltpu.sync_copy(datab.at[slot], acc.at[idxb.at[slot * 2 + 1]],
                            add=True)

            @pl.when(i + 2 < n)
            def _():
                pltpu.async_copy(epk_hbm.at[g + 2],
                                 idxb.at[pl.ds(slot * 2, 2)], si.at[slot])
            return _

        lax.fori_loop(0, n, body, None)

        plsc.subcore_barrier()
        pltpu.sync_copy(acc.at[pl.ds(s * STRIPE, STRIPE)],
                        parts_out.at[c, pl.ds(s * STRIPE, STRIPE)])

    return sc_agg


def _tc_dense_body(x_ref, parts_ref, w1_ref, b1_ref, w2_ref, b2_ref,
                   w3_ref, b3_ref, w4_ref, b4_ref, enc_ref, dec_ref):
    xs = x_ref[...]
    p = parts_ref[0] + parts_ref[1]
    cnt = p[:, D_FEAT:D_FEAT + 1]
    agg = p[:, :D_FEAT] / jnp.maximum(cnt, 1.0)
    col = lax.broadcasted_iota(jnp.int32, xs.shape, 1)
    xz = jnp.where(col == 0, 0.0, xs)
    aggz = jnp.where(col == 0, 0.0, agg)
    w1 = w1_ref[...]
    h = jnp.maximum(
        jnp.dot(xz, w1[:D_FEAT], preferred_element_type=jnp.float32)
        + jnp.dot(aggz, w1[D_FEAT:], preferred_element_type=jnp.float32)
        + b1_ref[...], 0.0)
    enc = jnp.dot(h, w2_ref[...], preferred_element_type=jnp.float32) + b2_ref[...]
    enc_ref[...] = enc
    h2 = jnp.maximum(
        jnp.dot(enc, w3_ref[...], preferred_element_type=jnp.float32)
        + b3_ref[...], 0.0)
    dec_ref[...] = (jnp.dot(h2, w4_ref[...], preferred_element_type=jnp.float32)
                    + b4_ref[...])


_TC_R = 1008


def _tc_dense(xp, parts, W_enc1, b_enc1, W_enc3, b_enc3,
              W_dec1, b_dec1, W_dec3, b_dec3):
    grid = (-(-N_NODES // _TC_R),)
    fixed = lambda i: (0, 0)
    enc, dec = pl.pallas_call(
        _tc_dense_body,
        grid=grid,
        in_specs=[
            pl.BlockSpec((_TC_R, D_FEAT), lambda i: (i, 0)),
            pl.BlockSpec((NC, _TC_R, AUGD), lambda i: (0, i, 0)),
            pl.BlockSpec((IN_DIM, H2), fixed),
            pl.BlockSpec((1, H2), fixed),
            pl.BlockSpec((H2, EMB), fixed),
            pl.BlockSpec((1, EMB), fixed),
            pl.BlockSpec((EMB, H2), fixed),
            pl.BlockSpec((1, H2), fixed),
            pl.BlockSpec((H2, IN_DIM), fixed),
            pl.BlockSpec((1, IN_DIM), fixed),
        ],
        out_specs=[
            pl.BlockSpec((_TC_R, EMB), lambda i: (i, 0)),
            pl.BlockSpec((_TC_R, IN_DIM), lambda i: (i, 0)),
        ],
        out_shape=[
            jax.ShapeDtypeStruct((N_NODES, EMB), jnp.float32),
            jax.ShapeDtypeStruct((N_NODES, IN_DIM), jnp.float32),
        ],
    )(xp, parts, W_enc1, b_enc1.reshape(1, H2), W_enc3, b_enc3.reshape(1, EMB),
      W_dec1, b_dec1.reshape(1, H2), W_dec3, b_dec3.reshape(1, IN_DIM))
    return enc, dec


def kernel(x, edge_index, W_enc1, b_enc1, W_enc3, b_enc3,
           W_dec1, b_dec1, W_dec3, b_dec3):
    xaug = jnp.concatenate(
        [x, jnp.ones((N_NODES, 1), jnp.float32),
         jnp.zeros((N_NODES, AUGD - D_FEAT - 1), jnp.float32)], axis=1)
    src = edge_index[0]
    dst = edge_index[1]
    pad = NEP - N_EDGES
    srcp = jnp.concatenate([src, jnp.zeros((pad,), jnp.int32)]).reshape(CH, 1, K)
    pad_dst = N_NODES + jnp.arange(pad, dtype=jnp.int32) % (ROWS - N_NODES)
    dstp = jnp.concatenate([dst, pad_dst]).reshape(CH, 1, K)
    epk = jnp.concatenate([srcp, dstp], axis=1)

    parts = _make_sc_agg()(xaug, epk)

    enc, dec = _tc_dense(x, parts, W_enc1, b_enc1, W_enc3, b_enc3,
                         W_dec1, b_dec1, W_dec3, b_dec3)
    return enc, dec
